# baseline (speedup 1.0000x reference)
import functools

import jax
import jax.numpy as jnp
from jax import lax
from jax.experimental import pallas as pl
from jax.experimental.pallas import tpu as pltpu
from jax.experimental.pallas import tpu_sc as plsc

B, S, D, E, H = 128, 32, 1024, 8, 4096
TR = 16
SROWS = 256
FBLK = 512
NF = H // FBLK


def _gather_mean_sc(tokens_flat, emb_table):
    info = plsc.get_sparse_core_info()
    nc, ns = info.num_cores, info.num_subcores
    nw = nc * ns
    rows_per_w = B // nw
    mesh = plsc.VectorSubcoreMesh(core_axis_name="c", subcore_axis_name="s")

    @functools.partial(
        pl.kernel,
        mesh=mesh,
        out_type=jax.ShapeDtypeStruct((B, D), jnp.float32),
        scratch_types=[
            pltpu.VMEM((S,), jnp.int32),
            pltpu.VMEM((S, D), jnp.float32),
            pltpu.VMEM((D,), jnp.float32),
            pltpu.SemaphoreType.DMA,
        ],
    )
    def k(tok_hbm, tab_hbm, out_hbm, idx_v, rows_v, acc_v, sem):
        wid = lax.axis_index("s") * nc + lax.axis_index("c")
        for r in range(rows_per_w):
            b = wid * rows_per_w + r
            pltpu.sync_copy(tok_hbm.at[pl.ds(b * S, S)], idx_v)
            pltpu.async_copy(tab_hbm.at[idx_v], rows_v, sem).wait()

            def cbody(c, _):
                sl = pl.ds(c * 16, 16)
                acc = rows_v[0, sl]
                for t in range(1, S):
                    acc = acc + rows_v[t, sl]
                acc_v[sl] = acc * (1.0 / S)
                return 0

            lax.fori_loop(0, D // 16, cbody, 0)
            pltpu.sync_copy(acc_v, out_hbm.at[b])

    return k(tokens_flat, emb_table)


def _router_body(x_ref, a_ref, scores_ref, winner_ref, oh_ref, pos_ref,
                 nt_ref, st_ref, sx_ref):
    x = x_ref[:]
    sc = lax.dot_general(x, a_ref[:], (((1,), (1,)), ((), ())),
                         preferred_element_type=jnp.float32)
    scores_ref[:] = sc
    m = jnp.max(sc, axis=1, keepdims=True)
    ii = lax.broadcasted_iota(jnp.int32, (B, E), 1).astype(jnp.float32)
    wf = jnp.min(jnp.where(sc >= m, ii, float(E)), axis=1, keepdims=True)
    winner_ref[:] = wf.astype(jnp.int32)
    oh = (ii == wf).astype(jnp.float32)
    oh_ref[:] = oh
    counts = jnp.sum(oh, axis=0, keepdims=True)
    nt = jnp.floor((counts + (TR - 1.0)) * (1.0 / TR))
    padded = nt * TR
    tri = (lax.broadcasted_iota(jnp.int32, (E, E), 0)
           < lax.broadcasted_iota(jnp.int32, (E, E), 1)).astype(jnp.float32)
    starts = lax.dot_general(padded, tri, (((1,), (0,)), ((), ())),
                             preferred_element_type=jnp.float32)
    ltri = (lax.broadcasted_iota(jnp.int32, (B, B), 0)
            >= lax.broadcasted_iota(jnp.int32, (B, B), 1)).astype(jnp.float32)
    cum = lax.dot_general(ltri, oh, (((1,), (0,)), ((), ())),
                          preferred_element_type=jnp.float32)
    rank = jnp.sum(cum * oh, axis=1, keepdims=True) - 1.0
    srow = lax.dot_general(oh, starts, (((1,), (1,)), ((), ())),
                           preferred_element_type=jnp.float32)
    pos = srow + rank
    pos_ref[:] = pos
    nt_ref[:] = nt.astype(jnp.int32)
    st_ref[:] = starts.astype(jnp.int32)
    pt = (pos == lax.broadcasted_iota(jnp.int32, (B, SROWS), 1
          ).astype(jnp.float32)).astype(jnp.float32)
    sx_ref[:] = lax.dot_general(pt, x, (((0,), (0,)), ((), ())),
                                preferred_element_type=jnp.float32)


def _router_tc(x, attractor):
    return pl.pallas_call(
        _router_body,
        out_shape=[
            jax.ShapeDtypeStruct((B, E), jnp.float32),
            jax.ShapeDtypeStruct((B, 1), jnp.int32),
            jax.ShapeDtypeStruct((B, E), jnp.float32),
            jax.ShapeDtypeStruct((B, 1), jnp.float32),
            jax.ShapeDtypeStruct((1, E), jnp.int32),
            jax.ShapeDtypeStruct((1, E), jnp.int32),
            jax.ShapeDtypeStruct((SROWS, D), jnp.float32),
        ],
    )(x, attractor)


def _ffn_body(nt_ref, st_ref, sx_ref, w1_ref, b1_ref, w2_ref, b2_ref,
              con_ref, oh_ref, pos_ref, out_ref, acc_ref):
    e = pl.program_id(0)
    f = pl.program_id(1)

    @pl.when((e == 0) & (f == 0))
    def _():
        acc_ref[:] = jnp.zeros((SROWS, D), jnp.float32)

    nt = nt_ref[0, e]
    st = st_ref[0, e]
    w1 = w1_ref[0]
    w2 = w2_ref[0]
    b1 = b1_ref[0]

    def tbody(t, _):
        r0 = pl.multiple_of(st + t * TR, TR)
        xs = sx_ref[pl.ds(r0, TR), :]
        h = jnp.maximum(
            lax.dot_general(xs, w1, (((1,), (0,)), ((), ())),
                            preferred_element_type=jnp.float32) + b1, 0.0)
        ct = lax.dot_general(h, w2, (((1,), (0,)), ((), ())),
                             preferred_element_type=jnp.float32)
        acc_ref[pl.ds(r0, TR), :] += ct
        return 0

    lax.fori_loop(0, nt, tbody, 0)

    @pl.when((e == E - 1) & (f == NF - 1))
    def _():
        pos = pos_ref[:]
        pt = (pos == lax.broadcasted_iota(jnp.int32, (B, SROWS), 1
              ).astype(jnp.float32)).astype(jnp.float32)
        outv = lax.dot_general(pt, acc_ref[:], (((1,), (0,)), ((), ())),
                               preferred_element_type=jnp.float32)
        oh = oh_ref[:]
        b2row = lax.dot_general(oh, b2_ref[:], (((1,), (0,)), ((), ())),
                                preferred_element_type=jnp.float32)
        cg = lax.dot_general(oh, con_ref[:], (((1,), (0,)), ((), ())),
                             preferred_element_type=jnp.float32)
        out_ref[:] = (outv + b2row) * cg


def _ffn_tc(nt, st, sx, W1, b1, W2, b2, con2d, oh, pos):
    grid = (E, NF)
    return pl.pallas_call(
        _ffn_body,
        grid=grid,
        in_specs=[
            pl.BlockSpec(memory_space=pltpu.SMEM),
            pl.BlockSpec(memory_space=pltpu.SMEM),
            pl.BlockSpec((SROWS, D), lambda e, f: (0, 0)),
            pl.BlockSpec((1, D, FBLK), lambda e, f: (e, 0, f)),
            pl.BlockSpec((1, 1, FBLK), lambda e, f: (e, 0, f)),
            pl.BlockSpec((1, FBLK, D), lambda e, f: (e, f, 0)),
            pl.BlockSpec((E, D), lambda e, f: (0, 0)),
            pl.BlockSpec((E, 1), lambda e, f: (0, 0)),
            pl.BlockSpec((B, E), lambda e, f: (0, 0)),
            pl.BlockSpec((B, 1), lambda e, f: (0, 0)),
        ],
        out_specs=pl.BlockSpec((B, D), lambda e, f: (0, 0)),
        out_shape=jax.ShapeDtypeStruct((B, D), jnp.float32),
        scratch_shapes=[pltpu.VMEM((SROWS, D), jnp.float32)],
        compiler_params=pltpu.CompilerParams(
            dimension_semantics=("arbitrary", "arbitrary")),
    )(nt, st, sx, W1, b1, W2, b2, con2d, oh, pos)


def kernel(tokens, emb_table, attractor, W1, b1, W2, b2, conatus):
    tokens_flat = tokens.reshape(B * S).astype(jnp.int32)
    emb_mean = _gather_mean_sc(tokens_flat, emb_table)
    scores, winner, oh, pos, nt, st, sx = _router_tc(emb_mean, attractor)
    out = _ffn_tc(nt, st, sx, W1, b1.reshape(E, 1, H), W2, b2,
                  conatus.reshape(E, 1), oh, pos)
    return out, scores, winner.reshape(B)


# speedup vs baseline: 1.2477x; 1.2477x over previous
import functools

import jax
import jax.numpy as jnp
from jax import lax
from jax.experimental import pallas as pl
from jax.experimental.pallas import tpu as pltpu
from jax.experimental.pallas import tpu_sc as plsc

B, S, D, E, H = 128, 32, 1024, 8, 4096
TR = 16
SROWS = 256
FBLK = 2048
NF = H // FBLK


def _gather_mean_sc(tokens_flat, emb_table):
    info = plsc.get_sparse_core_info()
    nc, ns = info.num_cores, info.num_subcores
    nw = nc * ns
    rows_per_w = B // nw
    mesh = plsc.VectorSubcoreMesh(core_axis_name="c", subcore_axis_name="s")

    @functools.partial(
        pl.kernel,
        mesh=mesh,
        out_type=jax.ShapeDtypeStruct((B, D), jnp.float32),
        scratch_types=[
            pltpu.VMEM((rows_per_w * S,), jnp.int32),
            pltpu.VMEM((3, S, D), jnp.float32),
            pltpu.VMEM((rows_per_w, D), jnp.float32),
            pltpu.SemaphoreType.DMA,
            pltpu.SemaphoreType.DMA,
            pltpu.SemaphoreType.DMA,
        ],
    )
    def k(tok_hbm, tab_hbm, out_hbm, idx_v, bufs, acc_v, sem0, sem1, sem2):
        wid = lax.axis_index("s") * nc + lax.axis_index("c")
        b0 = wid * rows_per_w
        pltpu.sync_copy(tok_hbm.at[pl.ds(b0 * S, rows_per_w * S)], idx_v)
        sems = (sem0, sem1, sem2)

        def start(r):
            pltpu.async_copy(tab_hbm.at[idx_v.at[pl.ds(r * S, S)]],
                             bufs.at[r % 3], sems[r % 3])

        def reduce_row(r):
            pltpu.make_async_copy(tab_hbm.at[idx_v.at[pl.ds(r * S, S)]],
                                  bufs.at[r % 3], sems[r % 3]).wait()

            def cbody(c, _):
                sl = pl.ds(c * 16, 16)
                acc = bufs[r % 3, 0, sl]
                for t in range(1, S):
                    acc = acc + bufs[r % 3, t, sl]
                acc_v[r, sl] = acc * (1.0 / S)
                return 0

            lax.fori_loop(0, D // 16, cbody, 0)

        start(0)
        start(1)
        start(2)
        for r in range(rows_per_w):
            reduce_row(r)
            if r + 3 < rows_per_w:
                start(r + 3)
        pltpu.sync_copy(acc_v, out_hbm.at[pl.ds(b0, rows_per_w)])

    return k(tokens_flat, emb_table)


def _router_body(x_ref, a_ref, scores_ref, winner_ref, oh_ref, pos_ref,
                 nt_ref, st_ref, sx_ref):
    x = x_ref[:]
    sc = lax.dot_general(x, a_ref[:], (((1,), (1,)), ((), ())),
                         preferred_element_type=jnp.float32)
    scores_ref[:] = sc
    m = jnp.max(sc, axis=1, keepdims=True)
    ii = lax.broadcasted_iota(jnp.int32, (B, E), 1).astype(jnp.float32)
    wf = jnp.min(jnp.where(sc >= m, ii, float(E)), axis=1, keepdims=True)
    winner_ref[:] = wf.astype(jnp.int32)
    oh = (ii == wf).astype(jnp.float32)
    oh_ref[:] = oh
    counts = jnp.sum(oh, axis=0, keepdims=True)
    nt = jnp.floor((counts + (TR - 1.0)) * (1.0 / TR))
    padded = nt * TR
    tri = (lax.broadcasted_iota(jnp.int32, (E, E), 0)
           < lax.broadcasted_iota(jnp.int32, (E, E), 1)).astype(jnp.float32)
    starts = lax.dot_general(padded, tri, (((1,), (0,)), ((), ())),
                             preferred_element_type=jnp.float32)
    ltri = (lax.broadcasted_iota(jnp.int32, (B, B), 0)
            >= lax.broadcasted_iota(jnp.int32, (B, B), 1)).astype(jnp.float32)
    cum = lax.dot_general(ltri, oh, (((1,), (0,)), ((), ())),
                          preferred_element_type=jnp.float32)
    rank = jnp.sum(cum * oh, axis=1, keepdims=True) - 1.0
    srow = lax.dot_general(oh, starts, (((1,), (1,)), ((), ())),
                           preferred_element_type=jnp.float32)
    pos = srow + rank
    pos_ref[:] = pos
    nt_ref[:] = nt.astype(jnp.int32)
    st_ref[:] = starts.astype(jnp.int32)
    pt = (pos == lax.broadcasted_iota(jnp.int32, (B, SROWS), 1
          ).astype(jnp.float32)).astype(jnp.float32)
    sx = lax.dot_general(pt, x, (((0,), (0,)), ((), ())),
                         preferred_element_type=jnp.float32)
    sx_ref[:] = sx.astype(jnp.bfloat16)


def _router_tc(x, attractor):
    return pl.pallas_call(
        _router_body,
        out_shape=[
            jax.ShapeDtypeStruct((B, E), jnp.float32),
            jax.ShapeDtypeStruct((B, 1), jnp.int32),
            jax.ShapeDtypeStruct((B, E), jnp.float32),
            jax.ShapeDtypeStruct((B, 1), jnp.float32),
            jax.ShapeDtypeStruct((1, E), jnp.int32),
            jax.ShapeDtypeStruct((1, E), jnp.int32),
            jax.ShapeDtypeStruct((SROWS, D), jnp.bfloat16),
        ],
    )(x, attractor)


def _ffn_body(nt_ref, st_ref, sx_ref, w1_ref, b1_ref, w2_ref, b2_ref,
              con_ref, oh_ref, pos_ref, out_ref, acc_ref):
    e = pl.program_id(0)
    f = pl.program_id(1)

    @pl.when((e == 0) & (f == 0))
    def _():
        acc_ref[:] = jnp.zeros((SROWS, D), jnp.float32)

    nt = nt_ref[0, e]
    st = st_ref[0, e]

    @pl.when(nt > 0)
    def _():
        b1 = b1_ref[0]

        def tbody(t, _):
            r0 = pl.multiple_of(st + t * TR, TR)
            xs = sx_ref[pl.ds(r0, TR), :].astype(jnp.float32)
            h = jnp.maximum(
                lax.dot_general(xs, w1_ref[0], (((1,), (0,)), ((), ())),
                                preferred_element_type=jnp.float32) + b1, 0.0)
            ct = lax.dot_general(h, w2_ref[0], (((1,), (0,)), ((), ())),
                                 preferred_element_type=jnp.float32)
            acc_ref[pl.ds(r0, TR), :] += ct
            return 0

        lax.fori_loop(0, nt, tbody, 0)

    @pl.when((e == E - 1) & (f == NF - 1))
    def _():
        pos = pos_ref[:]
        pt = (pos == lax.broadcasted_iota(jnp.int32, (B, SROWS), 1
              ).astype(jnp.float32)).astype(jnp.float32)
        outv = lax.dot_general(pt, acc_ref[:], (((1,), (0,)), ((), ())),
                               preferred_element_type=jnp.float32)
        oh = oh_ref[:]
        b2row = lax.dot_general(oh, b2_ref[:], (((1,), (0,)), ((), ())),
                                preferred_element_type=jnp.float32)
        cg = lax.dot_general(oh, con_ref[:], (((1,), (0,)), ((), ())),
                             preferred_element_type=jnp.float32)
        out_ref[:] = (outv + b2row) * cg


def _ffn_tc(nt, st, sx, W1, b1, W2, b2, con2d, oh, pos):
    grid = (E, NF)
    return pl.pallas_call(
        _ffn_body,
        grid=grid,
        in_specs=[
            pl.BlockSpec(memory_space=pltpu.SMEM),
            pl.BlockSpec(memory_space=pltpu.SMEM),
            pl.BlockSpec((SROWS, D), lambda e, f: (0, 0)),
            pl.BlockSpec((1, D, FBLK), lambda e, f: (e, 0, f)),
            pl.BlockSpec((1, 1, FBLK), lambda e, f: (e, 0, f)),
            pl.BlockSpec((1, FBLK, D), lambda e, f: (e, f, 0)),
            pl.BlockSpec((E, D), lambda e, f: (0, 0)),
            pl.BlockSpec((E, 1), lambda e, f: (0, 0)),
            pl.BlockSpec((B, E), lambda e, f: (0, 0)),
            pl.BlockSpec((B, 1), lambda e, f: (0, 0)),
        ],
        out_specs=pl.BlockSpec((B, D), lambda e, f: (0, 0)),
        out_shape=jax.ShapeDtypeStruct((B, D), jnp.float32),
        scratch_shapes=[pltpu.VMEM((SROWS, D), jnp.float32)],
        compiler_params=pltpu.CompilerParams(
            dimension_semantics=("arbitrary", "arbitrary"),
            vmem_limit_bytes=110 * 1024 * 1024),
    )(nt, st, sx, W1, b1, W2, b2, con2d, oh, pos)


def kernel(tokens, emb_table, attractor, W1, b1, W2, b2, conatus):
    tokens_flat = tokens.reshape(B * S).astype(jnp.int32)
    emb_mean = _gather_mean_sc(tokens_flat, emb_table)
    scores, winner, oh, pos, nt, st, sx = _router_tc(emb_mean, attractor)
    out = _ffn_tc(nt, st, sx, W1, b1.reshape(E, 1, H), W2, b2,
                  conatus.reshape(E, 1), oh, pos)
    return out, scores, winner.reshape(B)


# speedup vs baseline: 1.2612x; 1.0108x over previous
import functools

import jax
import jax.numpy as jnp
from jax import lax
from jax.experimental import pallas as pl
from jax.experimental.pallas import tpu as pltpu
from jax.experimental.pallas import tpu_sc as plsc

B, S, D, E, H = 128, 32, 1024, 8, 4096
TR = 16
SROWS = 256
FBLK = 2048
NF = H // FBLK


def _gather_mean_sc(tokens_flat, emb_table):
    info = plsc.get_sparse_core_info()
    nc, ns = info.num_cores, info.num_subcores
    nw = nc * ns
    rows_per_w = B // nw
    mesh = plsc.VectorSubcoreMesh(core_axis_name="c", subcore_axis_name="s")

    @functools.partial(
        pl.kernel,
        mesh=mesh,
        out_type=jax.ShapeDtypeStruct((B, D), jnp.float32),
        scratch_types=[
            pltpu.VMEM((rows_per_w * S,), jnp.int32),
            pltpu.VMEM((2, S, D), jnp.float32),
            pltpu.VMEM((rows_per_w, D), jnp.float32),
            pltpu.SemaphoreType.DMA,
            pltpu.SemaphoreType.DMA,
        ],
    )
    def k(tok_hbm, tab_hbm, out_hbm, idx_v, bufs, acc_v, sem0, sem1):
        wid = lax.axis_index("s") * nc + lax.axis_index("c")
        b0 = wid * rows_per_w
        pltpu.sync_copy(tok_hbm.at[pl.ds(b0 * S, rows_per_w * S)], idx_v)
        sems = (sem0, sem1)

        def start(r):
            pltpu.async_copy(tab_hbm.at[idx_v.at[pl.ds(r * S, S)]],
                             bufs.at[r % 2], sems[r % 2])

        def reduce_row(r):
            pltpu.make_async_copy(tab_hbm.at[idx_v.at[pl.ds(r * S, S)]],
                                  bufs.at[r % 2], sems[r % 2]).wait()

            def cbody(c, _):
                sl = pl.ds(c * 16, 16)
                acc = bufs[r % 2, 0, sl]
                for t in range(1, S):
                    acc = acc + bufs[r % 2, t, sl]
                acc_v[r, sl] = acc * (1.0 / S)
                return 0

            lax.fori_loop(0, D // 16, cbody, 0)

        start(0)
        start(1)
        for r in range(rows_per_w):
            reduce_row(r)
            if r + 2 < rows_per_w:
                start(r + 2)
        pltpu.sync_copy(acc_v, out_hbm.at[pl.ds(b0, rows_per_w)])

    return k(tokens_flat, emb_table)


def _router_body(x_ref, a_ref, scores_ref, winner_ref, oh_ref, pos_ref,
                 nt_ref, st_ref, sx_ref):
    x = x_ref[:]
    sc = lax.dot_general(x, a_ref[:], (((1,), (1,)), ((), ())),
                         preferred_element_type=jnp.float32)
    scores_ref[:] = sc
    m = jnp.max(sc, axis=1, keepdims=True)
    ii = lax.broadcasted_iota(jnp.int32, (B, E), 1).astype(jnp.float32)
    wf = jnp.min(jnp.where(sc >= m, ii, float(E)), axis=1, keepdims=True)
    winner_ref[:] = wf.astype(jnp.int32)
    oh = (ii == wf).astype(jnp.float32)
    oh_ref[:] = oh
    counts = jnp.sum(oh, axis=0, keepdims=True)
    nt = jnp.floor((counts + (TR - 1.0)) * (1.0 / TR))
    padded = nt * TR
    tri = (lax.broadcasted_iota(jnp.int32, (E, E), 0)
           < lax.broadcasted_iota(jnp.int32, (E, E), 1)).astype(jnp.float32)
    starts = lax.dot_general(padded, tri, (((1,), (0,)), ((), ())),
                             preferred_element_type=jnp.float32)
    ltri = (lax.broadcasted_iota(jnp.int32, (B, B), 0)
            >= lax.broadcasted_iota(jnp.int32, (B, B), 1)).astype(jnp.float32)
    cum = lax.dot_general(ltri, oh, (((1,), (0,)), ((), ())),
                          preferred_element_type=jnp.float32)
    rank = jnp.sum(cum * oh, axis=1, keepdims=True) - 1.0
    srow = lax.dot_general(oh, starts, (((1,), (1,)), ((), ())),
                           preferred_element_type=jnp.float32)
    pos = srow + rank
    pos_ref[:] = pos
    nt_ref[:] = nt.astype(jnp.int32)
    st_ref[:] = starts.astype(jnp.int32)
    pt = (pos == lax.broadcasted_iota(jnp.int32, (B, SROWS), 1
          ).astype(jnp.float32)).astype(jnp.float32)
    sx = lax.dot_general(pt, x, (((0,), (0,)), ((), ())),
                         preferred_element_type=jnp.float32)
    sx_ref[:] = sx.astype(jnp.bfloat16)


def _router_tc(x, attractor):
    return pl.pallas_call(
        _router_body,
        out_shape=[
            jax.ShapeDtypeStruct((B, E), jnp.float32),
            jax.ShapeDtypeStruct((B, 1), jnp.int32),
            jax.ShapeDtypeStruct((B, E), jnp.float32),
            jax.ShapeDtypeStruct((B, 1), jnp.float32),
            jax.ShapeDtypeStruct((1, E), jnp.int32),
            jax.ShapeDtypeStruct((1, E), jnp.int32),
            jax.ShapeDtypeStruct((SROWS, D), jnp.bfloat16),
        ],
    )(x, attractor)


def _ffn_body(nt_ref, st_ref, sx_ref, w1_ref, b1_ref, w2_ref, b2_ref,
              con_ref, oh_ref, pos_ref, out_ref, acc_ref):
    e = pl.program_id(0)
    f = pl.program_id(1)

    @pl.when((e == 0) & (f == 0))
    def _():
        acc_ref[:] = jnp.zeros((SROWS, D), jnp.float32)

    nt = nt_ref[0, e]
    st = st_ref[0, e]

    @pl.when(nt > 0)
    def _():
        b1 = b1_ref[0]

        def tbody(t, _):
            r0 = pl.multiple_of(st + t * TR, TR)
            xs = sx_ref[pl.ds(r0, TR), :].astype(jnp.float32)
            h = jnp.maximum(
                lax.dot_general(xs, w1_ref[0], (((1,), (0,)), ((), ())),
                                preferred_element_type=jnp.float32) + b1, 0.0)
            ct = lax.dot_general(h, w2_ref[0], (((1,), (0,)), ((), ())),
                                 preferred_element_type=jnp.float32)
            acc_ref[pl.ds(r0, TR), :] += ct
            return 0

        lax.fori_loop(0, nt, tbody, 0)

    @pl.when((e == E - 1) & (f == NF - 1))
    def _():
        pos = pos_ref[:]
        pt = (pos == lax.broadcasted_iota(jnp.int32, (B, SROWS), 1
              ).astype(jnp.float32)).astype(jnp.float32)
        outv = lax.dot_general(pt, acc_ref[:], (((1,), (0,)), ((), ())),
                               preferred_element_type=jnp.float32)
        oh = oh_ref[:]
        b2row = lax.dot_general(oh, b2_ref[:], (((1,), (0,)), ((), ())),
                                preferred_element_type=jnp.float32)
        cg = lax.dot_general(oh, con_ref[:], (((1,), (0,)), ((), ())),
                             preferred_element_type=jnp.float32)
        out_ref[:] = (outv + b2row) * cg


def _ffn_tc(nt, st, sx, W1, b1, W2, b2, con2d, oh, pos):
    grid = (E, NF)
    return pl.pallas_call(
        _ffn_body,
        grid=grid,
        in_specs=[
            pl.BlockSpec(memory_space=pltpu.SMEM),
            pl.BlockSpec(memory_space=pltpu.SMEM),
            pl.BlockSpec((SROWS, D), lambda e, f: (0, 0)),
            pl.BlockSpec((1, D, FBLK), lambda e, f: (e, 0, f)),
            pl.BlockSpec((1, 1, FBLK), lambda e, f: (e, 0, f)),
            pl.BlockSpec((1, FBLK, D), lambda e, f: (e, f, 0)),
            pl.BlockSpec((E, D), lambda e, f: (0, 0)),
            pl.BlockSpec((E, 1), lambda e, f: (0, 0)),
            pl.BlockSpec((B, E), lambda e, f: (0, 0)),
            pl.BlockSpec((B, 1), lambda e, f: (0, 0)),
        ],
        out_specs=pl.BlockSpec((B, D), lambda e, f: (0, 0)),
        out_shape=jax.ShapeDtypeStruct((B, D), jnp.float32),
        scratch_shapes=[pltpu.VMEM((SROWS, D), jnp.float32)],
        compiler_params=pltpu.CompilerParams(
            dimension_semantics=("arbitrary", "arbitrary"),
            vmem_limit_bytes=110 * 1024 * 1024),
    )(nt, st, sx, W1, b1, W2, b2, con2d, oh, pos)


def kernel(tokens, emb_table, attractor, W1, b1, W2, b2, conatus):
    tokens_flat = tokens.reshape(B * S).astype(jnp.int32)
    emb_mean = _gather_mean_sc(tokens_flat, emb_table)
    scores, winner, oh, pos, nt, st, sx = _router_tc(emb_mean, attractor)
    out = _ffn_tc(nt, st, sx, W1, b1.reshape(E, 1, H), W2, b2,
                  conatus.reshape(E, 1), oh, pos)
    return out, scores, winner.reshape(B)


# speedup vs baseline: 1.2981x; 1.0293x over previous
import functools

import jax
import jax.numpy as jnp
from jax import lax
from jax.experimental import pallas as pl
from jax.experimental.pallas import tpu as pltpu
from jax.experimental.pallas import tpu_sc as plsc

B, S, D, E, H = 128, 32, 1024, 8, 4096
TR = 16
SROWS = 256
FBLK = 2048
NF = H // FBLK


def _gather_mean_sc(tokens_flat, emb_table):
    info = plsc.get_sparse_core_info()
    nc, ns = info.num_cores, info.num_subcores
    nw = nc * ns
    rows_per_w = B // nw
    mesh = plsc.VectorSubcoreMesh(core_axis_name="c", subcore_axis_name="s")

    @functools.partial(
        pl.kernel,
        mesh=mesh,
        out_type=jax.ShapeDtypeStruct((B, D), jnp.float32),
        scratch_types=[
            pltpu.VMEM((rows_per_w * S,), jnp.int32),
            pltpu.VMEM((2, S, D), jnp.float32),
            pltpu.VMEM((rows_per_w, D), jnp.float32),
            pltpu.SemaphoreType.DMA,
            pltpu.SemaphoreType.DMA,
        ],
    )
    def k(tok_hbm, tab_hbm, out_hbm, idx_v, bufs, acc_v, sem0, sem1):
        wid = lax.axis_index("s") * nc + lax.axis_index("c")
        b0 = wid * rows_per_w
        pltpu.sync_copy(tok_hbm.at[pl.ds(b0 * S, rows_per_w * S)], idx_v)
        sems = (sem0, sem1)

        def start(r):
            pltpu.async_copy(tab_hbm.at[idx_v.at[pl.ds(r * S, S)]],
                             bufs.at[r % 2], sems[r % 2])

        def reduce_row(r):
            pltpu.make_async_copy(tab_hbm.at[idx_v.at[pl.ds(r * S, S)]],
                                  bufs.at[r % 2], sems[r % 2]).wait()

            def cbody(c, _):
                sl = pl.ds(c * 16, 16)
                vals = [bufs[r % 2, t, sl] for t in range(S)]
                while len(vals) > 1:
                    vals = [vals[i] + vals[i + 1]
                            for i in range(0, len(vals), 2)]
                acc_v[r, sl] = vals[0] * (1.0 / S)
                return 0

            lax.fori_loop(0, D // 16, cbody, 0)

        start(0)
        start(1)
        for r in range(rows_per_w):
            reduce_row(r)
            if r + 2 < rows_per_w:
                start(r + 2)
        pltpu.sync_copy(acc_v, out_hbm.at[pl.ds(b0, rows_per_w)])

    return k(tokens_flat, emb_table)


def _router_body(x_ref, a_ref, scores_ref, winner_ref, oh_ref, pos_ref,
                 nt_ref, st_ref, sx_ref):
    x = x_ref[:]
    sc = lax.dot_general(x, a_ref[:], (((1,), (1,)), ((), ())),
                         preferred_element_type=jnp.float32)
    scores_ref[:] = sc
    m = jnp.max(sc, axis=1, keepdims=True)
    ii = lax.broadcasted_iota(jnp.int32, (B, E), 1).astype(jnp.float32)
    wf = jnp.min(jnp.where(sc >= m, ii, float(E)), axis=1, keepdims=True)
    winner_ref[:] = wf.astype(jnp.int32)
    oh = (ii == wf).astype(jnp.float32)
    oh_ref[:] = oh
    counts = jnp.sum(oh, axis=0, keepdims=True)
    nt = jnp.floor((counts + (TR - 1.0)) * (1.0 / TR))
    padded = nt * TR
    tri = (lax.broadcasted_iota(jnp.int32, (E, E), 0)
           < lax.broadcasted_iota(jnp.int32, (E, E), 1)).astype(jnp.float32)
    starts = lax.dot_general(padded, tri, (((1,), (0,)), ((), ())),
                             preferred_element_type=jnp.float32)
    ltri = (lax.broadcasted_iota(jnp.int32, (B, B), 0)
            >= lax.broadcasted_iota(jnp.int32, (B, B), 1)).astype(jnp.float32)
    cum = lax.dot_general(ltri, oh, (((1,), (0,)), ((), ())),
                          preferred_element_type=jnp.float32)
    rank = jnp.sum(cum * oh, axis=1, keepdims=True) - 1.0
    srow = lax.dot_general(oh, starts, (((1,), (1,)), ((), ())),
                           preferred_element_type=jnp.float32)
    pos = srow + rank
    pos_ref[:] = pos
    nt_ref[:] = nt.astype(jnp.int32)
    st_ref[:] = starts.astype(jnp.int32)
    pt = (pos == lax.broadcasted_iota(jnp.int32, (B, SROWS), 1
          ).astype(jnp.float32)).astype(jnp.float32)
    sx = lax.dot_general(pt, x, (((0,), (0,)), ((), ())),
                         preferred_element_type=jnp.float32)
    sx_ref[:] = sx.astype(jnp.bfloat16)


def _router_tc(x, attractor):
    return pl.pallas_call(
        _router_body,
        out_shape=[
            jax.ShapeDtypeStruct((B, E), jnp.float32),
            jax.ShapeDtypeStruct((B, 1), jnp.int32),
            jax.ShapeDtypeStruct((B, E), jnp.float32),
            jax.ShapeDtypeStruct((B, 1), jnp.float32),
            jax.ShapeDtypeStruct((1, E), jnp.int32),
            jax.ShapeDtypeStruct((1, E), jnp.int32),
            jax.ShapeDtypeStruct((SROWS, D), jnp.bfloat16),
        ],
    )(x, attractor)


def _ffn_body(nt_ref, st_ref, sx_ref, w1_ref, b1_ref, w2_ref, b2_ref,
              con_ref, oh_ref, pos_ref, out_ref, acc_ref):
    e = pl.program_id(0)
    f = pl.program_id(1)

    @pl.when((e == 0) & (f == 0))
    def _():
        acc_ref[:] = jnp.zeros((SROWS, D), jnp.float32)

    nt = nt_ref[0, e]
    st = st_ref[0, e]

    @pl.when(nt > 0)
    def _():
        b1 = b1_ref[0]

        def tbody(t, _):
            r0 = pl.multiple_of(st + t * TR, TR)
            xs = sx_ref[pl.ds(r0, TR), :].astype(jnp.float32)
            h = jnp.maximum(
                lax.dot_general(xs, w1_ref[0], (((1,), (0,)), ((), ())),
                                preferred_element_type=jnp.float32) + b1, 0.0)
            ct = lax.dot_general(h, w2_ref[0], (((1,), (0,)), ((), ())),
                                 preferred_element_type=jnp.float32)
            acc_ref[pl.ds(r0, TR), :] += ct
            return 0

        lax.fori_loop(0, nt, tbody, 0)

    @pl.when((e == E - 1) & (f == NF - 1))
    def _():
        pos = pos_ref[:]
        pt = (pos == lax.broadcasted_iota(jnp.int32, (B, SROWS), 1
              ).astype(jnp.float32)).astype(jnp.float32)
        outv = lax.dot_general(pt, acc_ref[:], (((1,), (0,)), ((), ())),
                               preferred_element_type=jnp.float32)
        oh = oh_ref[:]
        b2row = lax.dot_general(oh, b2_ref[:], (((1,), (0,)), ((), ())),
                                preferred_element_type=jnp.float32)
        cg = lax.dot_general(oh, con_ref[:], (((1,), (0,)), ((), ())),
                             preferred_element_type=jnp.float32)
        out_ref[:] = (outv + b2row) * cg


def _ffn_tc(nt, st, sx, W1, b1, W2, b2, con2d, oh, pos):
    grid = (E, NF)
    return pl.pallas_call(
        _ffn_body,
        grid=grid,
        in_specs=[
            pl.BlockSpec(memory_space=pltpu.SMEM),
            pl.BlockSpec(memory_space=pltpu.SMEM),
            pl.BlockSpec((SROWS, D), lambda e, f: (0, 0)),
            pl.BlockSpec((1, D, FBLK), lambda e, f: (e, 0, f)),
            pl.BlockSpec((1, 1, FBLK), lambda e, f: (e, 0, f)),
            pl.BlockSpec((1, FBLK, D), lambda e, f: (e, f, 0)),
            pl.BlockSpec((E, D), lambda e, f: (0, 0)),
            pl.BlockSpec((E, 1), lambda e, f: (0, 0)),
            pl.BlockSpec((B, E), lambda e, f: (0, 0)),
            pl.BlockSpec((B, 1), lambda e, f: (0, 0)),
        ],
        out_specs=pl.BlockSpec((B, D), lambda e, f: (0, 0)),
        out_shape=jax.ShapeDtypeStruct((B, D), jnp.float32),
        scratch_shapes=[pltpu.VMEM((SROWS, D), jnp.float32)],
        compiler_params=pltpu.CompilerParams(
            dimension_semantics=("arbitrary", "arbitrary"),
            vmem_limit_bytes=110 * 1024 * 1024),
    )(nt, st, sx, W1, b1, W2, b2, con2d, oh, pos)


def kernel(tokens, emb_table, attractor, W1, b1, W2, b2, conatus):
    tokens_flat = tokens.reshape(B * S).astype(jnp.int32)
    emb_mean = _gather_mean_sc(tokens_flat, emb_table)
    scores, winner, oh, pos, nt, st, sx = _router_tc(emb_mean, attractor)
    out = _ffn_tc(nt, st, sx, W1, b1.reshape(E, 1, H), W2, b2,
                  conatus.reshape(E, 1), oh, pos)
    return out, scores, winner.reshape(B)
